# baseline (speedup 1.0000x reference)
import functools

import jax
import jax.numpy as jnp
from jax import lax
from jax.experimental import pallas as pl
from jax.experimental.pallas import tpu as pltpu
from jax.experimental.pallas import tpu_sc as plsc

VOCAB = 1000000
D = 32
B = 4096
S = 50
L = 16
NC, NS = 2, 16
NW = NC * NS
ROWS_W = B // NW
ROWS_P = 64
NPASS = ROWS_W // ROWS_P
TOK_P = ROWS_P * S
KW = (S + L - 1) // L

TILE_V = 128
NVT = 999936 // TILE_V
G = 4
NB_ALL = NVT // G
NSLOT = 61
TAIL_V0 = NVT * TILE_V
TAIL_N = VOCAB - TAIL_V0
BATCH_W = G * TILE_V * D


def _tbody(word_t, tail_flat, out_flat, tin_a, tin_b, tout_a, tout_b,
           isem, osem):
    wid = lax.axis_index("s") * NC + lax.axis_index("c")
    iota = lax.iota(jnp.int32, L)

    @pl.when(wid == 0)
    def _():
        pltpu.sync_copy(tail_flat, tout_a.at[pl.ds(0, TAIL_N * D)])
        pltpu.sync_copy(tout_a.at[pl.ds(0, TAIL_N * D)],
                        out_flat.at[pl.ds(TAIL_V0 * D, TAIL_N * D)])

    def fire_in(slot, tin):
        v0 = (wid + NW * slot) * (G * TILE_V)
        for dt in range(4):
            pltpu.async_copy(
                word_t.at[pl.ds(dt * 8, 8), pl.ds(v0, G * TILE_V)],
                tin.at[pl.ds(dt * 8, 8), :], isem)

    def wait_in(tin):
        for dt in range(4):
            pltpu.make_async_copy(
                word_t.at[pl.ds(0, 8), pl.ds(0, G * TILE_V)],
                tin.at[pl.ds(dt * 8, 8), :], isem).wait()

    iota32 = iota * D

    def transpose(tin, tout):
        @plsc.parallel_loop(0, G * TILE_V // L, unroll=2)
        def _(vg):
            vgoff = vg * L
            base = iota32 + lax.broadcast(vg * (L * D), (L,))
            vals = [tin[d, pl.ds(vgoff, L)] for d in range(32)]
            for d in range(32):
                plsc.store_scatter(tout, [base + d], vals[d])

    def fire_out(slot, tout):
        bi = wid + NW * slot
        pltpu.async_copy(tout, out_flat.at[pl.ds(bi * BATCH_W, BATCH_W)], osem)

    def wait_out(tout):
        pltpu.make_async_copy(
            out_flat.at[pl.ds(0, BATCH_W)], tout, osem).wait()

    fire_in(0, tin_a)

    def step(k, carry):
        slot_a = 2 * k
        slot_b = 2 * k + 1
        fire_in(slot_b, tin_b)
        wait_in(tin_a)

        @pl.when(k > 0)
        def _():
            wait_out(tout_a)

        transpose(tin_a, tout_a)
        fire_out(slot_a, tout_a)
        fire_in(slot_a + 2, tin_a)
        wait_in(tin_b)

        @pl.when(k > 0)
        def _():
            wait_out(tout_b)

        transpose(tin_b, tout_b)
        fire_out(slot_b, tout_b)
        return carry

    lax.fori_loop(0, (NSLOT - 1) // 2, step, 0)

    wait_in(tin_a)
    wait_out(tout_a)
    transpose(tin_a, tout_a)
    fire_out(NSLOT - 1, tout_a)

    @pl.when(wid == NW - 1)
    def _():
        wait_out(tout_b)
        v0 = (NB_ALL - 1) * (G * TILE_V)
        for dt in range(4):
            pltpu.async_copy(
                word_t.at[pl.ds(dt * 8, 8), pl.ds(v0, G * TILE_V)],
                tin_b.at[pl.ds(dt * 8, 8), :], isem)
        wait_in(tin_b)
        transpose(tin_b, tout_b)
        pltpu.sync_copy(tout_b,
                        out_flat.at[pl.ds((NB_ALL - 1) * BATCH_W, BATCH_W)])

    wait_out(tout_a)
    @pl.when(wid != NW - 1)
    def _():
        wait_out(tout_b)


@jax.jit
def _transpose_run(word_t, tail_flat):
    mesh = plsc.VectorSubcoreMesh(core_axis_name="c", subcore_axis_name="s")
    return pl.kernel(
        _tbody,
        out_type=jax.ShapeDtypeStruct((VOCAB * D,), jnp.float32),
        mesh=mesh,
        scratch_types=[
            pltpu.VMEM((32, G * TILE_V), jnp.float32),
            pltpu.VMEM((32, G * TILE_V), jnp.float32),
            pltpu.VMEM((BATCH_W,), jnp.float32),
            pltpu.VMEM((BATCH_W,), jnp.float32),
            pltpu.SemaphoreType.DMA,
            pltpu.SemaphoreType.DMA,
        ],
        compiler_params=pltpu.CompilerParams(
            needs_layout_passes=False, use_tc_tiling_on_sc=True),
    )(word_t, tail_flat)


def _body(desc_t, word_hbm, weight_hbm, out_hbm,
          idx_v, emb_v, w_v, wexp_v, out_v, gsem, wsem):
    wid = lax.axis_index("s") * NC + lax.axis_index("c")
    iota = lax.iota(jnp.int32, L)
    col0 = wid * ROWS_W

    pltpu.sync_copy(desc_t.at[:, pl.ds(col0, ROWS_W)], idx_v)

    for p in range(NPASS):
        copies = []
        for j in range(S):
            ids = idx_v.at[j, pl.ds(p * ROWS_P, ROWS_P)]
            copies.append(pltpu.async_copy(
                word_hbm.at[ids], emb_v.at[pl.ds(j * ROWS_P, ROWS_P), :], gsem))
            copies.append(pltpu.async_copy(
                weight_hbm.at[ids], w_v.at[pl.ds(j * ROWS_P, ROWS_P)], wsem))
        for c in copies:
            c.wait()

        def row_body(r, _):
            wvecs = []
            for k in range(KW):
                idxs = jnp.minimum(k * L + iota, S - 1) * ROWS_P + r
                wvecs.append(plsc.load_gather(w_v, [idxs]))
            masks = [(k * L + iota) < S for k in range(KW)]
            mvec = jnp.where(masks[0], wvecs[0], -1e30)
            for k in range(1, KW):
                mvec = jnp.maximum(mvec, jnp.where(masks[k], wvecs[k], -1e30))
            mx = jnp.max(mvec)
            svec = jnp.zeros((L,), jnp.float32)
            evecs = []
            for k in range(KW):
                e_k = jnp.where(masks[k], jnp.exp(wvecs[k] - mx), 0.0)
                evecs.append(e_k)
                svec = svec + e_k
            inv = jnp.ones((L,), jnp.float32) / lax.broadcast(jnp.sum(svec), (L,))
            wbase = r * (KW * L)
            for k in range(KW):
                wexp_v[pl.ds(wbase + k * L, L)] = evecs[k] * inv
            acc0 = jnp.zeros((L,), jnp.float32)
            acc1 = jnp.zeros((L,), jnp.float32)
            for j in range(S):
                wb = plsc.load_gather(wexp_v, [lax.broadcast(wbase + j, (L,))])
                acc0 = acc0 + wb * emb_v[j * ROWS_P + r, pl.ds(0, L)]
                acc1 = acc1 + wb * emb_v[j * ROWS_P + r, pl.ds(L, L)]
            out_v[r, pl.ds(0, L)] = acc0
            out_v[r, pl.ds(L, L)] = acc1
            return _

        lax.fori_loop(0, ROWS_P, row_body, 0)

        pltpu.sync_copy(out_v, out_hbm.at[pl.ds(col0 + p * ROWS_P, ROWS_P), :])


@jax.jit
def _run(desc_t, word_rm, weight_flat):
    mesh = plsc.VectorSubcoreMesh(core_axis_name="c", subcore_axis_name="s")
    return pl.kernel(
        _body,
        out_type=jax.ShapeDtypeStruct((B, D), jnp.float32),
        mesh=mesh,
        scratch_types=[
            pltpu.VMEM((S, ROWS_W), jnp.int32),
            pltpu.VMEM((TOK_P, D), jnp.float32),
            pltpu.VMEM((TOK_P,), jnp.float32),
            pltpu.VMEM((ROWS_P * KW * L,), jnp.float32),
            pltpu.VMEM((ROWS_P, D), jnp.float32),
            pltpu.SemaphoreType.DMA,
            pltpu.SemaphoreType.DMA,
        ],
        compiler_params=pltpu.CompilerParams(
            needs_layout_passes=False, use_tc_tiling_on_sc=False),
    )(desc_t, word_rm, weight_flat)


def kernel(desc, word_table, weight_table):
    tail_flat = word_table[TAIL_V0:, :].reshape(TAIL_N * D)
    wt_flat = _transpose_run(word_table.T, tail_flat)
    wt_rm = wt_flat.reshape(VOCAB, D)
    return _run(desc.T, wt_rm, weight_table.reshape(VOCAB))


# speedup vs baseline: 1.1944x; 1.1944x over previous
import functools

import jax
import jax.numpy as jnp
from jax import lax
from jax.experimental import pallas as pl
from jax.experimental.pallas import tpu as pltpu
from jax.experimental.pallas import tpu_sc as plsc

VOCAB = 1000000
D = 32
B = 4096
S = 50
L = 16
NC, NS = 2, 16
NW = NC * NS
ROWS_W = B // NW
ROWS_P = 64
NPASS = ROWS_W // ROWS_P
TOK_P = ROWS_P * S
KW = (S + L - 1) // L

TILE_V = 128
NVT = 999936 // TILE_V
G = 4
NB_ALL = NVT // G
NSLOT = 61
TAIL_V0 = NVT * TILE_V
TAIL_N = VOCAB - TAIL_V0
BATCH_W = G * TILE_V * D


def _tbody(word_t, tail_flat, out_flat, tin_a, tin_b, tout_a, tout_b,
           isem, osem):
    wid = lax.axis_index("s") * NC + lax.axis_index("c")
    iota = lax.iota(jnp.int32, L)

    @pl.when(wid == 0)
    def _():
        pltpu.sync_copy(tail_flat, tout_a.at[pl.ds(0, TAIL_N * D)])
        pltpu.sync_copy(tout_a.at[pl.ds(0, TAIL_N * D)],
                        out_flat.at[pl.ds(TAIL_V0 * D, TAIL_N * D)])

    def fire_in(slot, tin):
        v0 = (wid + NW * slot) * (G * TILE_V)
        pltpu.async_copy(
            word_t.at[:, pl.ds(v0, G * TILE_V)], tin, isem)

    def wait_in(tin):
        pltpu.make_async_copy(
            word_t.at[:, pl.ds(0, G * TILE_V)], tin, isem).wait()

    cvec0 = iota * (G * TILE_V)
    cvec1 = (iota + L) * (G * TILE_V)
    zerov = jnp.zeros((L,), jnp.int32)

    def transpose(tin, tout):
        @plsc.parallel_loop(0, G * 32, unroll=8)
        def _(tq):
            base = tq * TILE_V
            colb = lax.shift_left(lax.shift_right_logical(tq, 5), 7) \
                + lax.shift_left(jnp.bitwise_and(tq, 31), 2)
            for j in range(4):
                cols = lax.broadcast(colb + j, (L,))
                for c0 in (0, 16):
                    g = plsc.load_gather(tin, [c0 + iota, cols])
                    tout[pl.ds(base + j * D + c0, L)] = g

    def fire_out(slot, tout):
        bi = wid + NW * slot
        pltpu.async_copy(tout, out_flat.at[pl.ds(bi * BATCH_W, BATCH_W)], osem)

    def wait_out(tout):
        pltpu.make_async_copy(
            out_flat.at[pl.ds(0, BATCH_W)], tout, osem).wait()

    fire_in(0, tin_a)

    def step(k, carry):
        slot_a = 2 * k
        slot_b = 2 * k + 1
        fire_in(slot_b, tin_b)
        wait_in(tin_a)

        @pl.when(k > 0)
        def _():
            wait_out(tout_a)

        transpose(tin_a, tout_a)
        fire_out(slot_a, tout_a)
        fire_in(slot_a + 2, tin_a)
        wait_in(tin_b)

        @pl.when(k > 0)
        def _():
            wait_out(tout_b)

        transpose(tin_b, tout_b)
        fire_out(slot_b, tout_b)
        return carry

    lax.fori_loop(0, (NSLOT - 1) // 2, step, 0)

    wait_in(tin_a)
    wait_out(tout_a)
    transpose(tin_a, tout_a)
    fire_out(NSLOT - 1, tout_a)

    @pl.when(wid == NW - 1)
    def _():
        wait_out(tout_b)
        v0 = (NB_ALL - 1) * (G * TILE_V)
        pltpu.async_copy(
            word_t.at[:, pl.ds(v0, G * TILE_V)], tin_b, isem)
        wait_in(tin_b)
        transpose(tin_b, tout_b)
        pltpu.sync_copy(tout_b,
                        out_flat.at[pl.ds((NB_ALL - 1) * BATCH_W, BATCH_W)])

    wait_out(tout_a)
    @pl.when(wid != NW - 1)
    def _():
        wait_out(tout_b)


@jax.jit
def _transpose_run(word_t, tail_flat):
    mesh = plsc.VectorSubcoreMesh(core_axis_name="c", subcore_axis_name="s")
    return pl.kernel(
        _tbody,
        out_type=jax.ShapeDtypeStruct((VOCAB * D,), jnp.float32),
        mesh=mesh,
        scratch_types=[
            pltpu.VMEM((32, G * TILE_V), jnp.float32),
            pltpu.VMEM((32, G * TILE_V), jnp.float32),
            pltpu.VMEM((BATCH_W,), jnp.float32),
            pltpu.VMEM((BATCH_W,), jnp.float32),
            pltpu.SemaphoreType.DMA,
            pltpu.SemaphoreType.DMA,
        ],
        compiler_params=pltpu.CompilerParams(
            needs_layout_passes=False, use_tc_tiling_on_sc=True),
    )(word_t, tail_flat)


def _body(desc_t, word_hbm, weight_hbm, out_hbm,
          idx_v, emb_v, w_v, wexp_v, out_v, gsem, wsem):
    wid = lax.axis_index("s") * NC + lax.axis_index("c")
    iota = lax.iota(jnp.int32, L)
    col0 = wid * ROWS_W

    pltpu.sync_copy(desc_t.at[:, pl.ds(col0, ROWS_W)], idx_v)

    for p in range(NPASS):
        copies = []
        for j in range(S):
            ids = idx_v.at[j, pl.ds(p * ROWS_P, ROWS_P)]
            copies.append(pltpu.async_copy(
                word_hbm.at[ids], emb_v.at[pl.ds(j * ROWS_P, ROWS_P), :], gsem))
            copies.append(pltpu.async_copy(
                weight_hbm.at[ids], w_v.at[pl.ds(j * ROWS_P, ROWS_P)], wsem))
        for c in copies:
            c.wait()

        def row_body(r, _):
            wvecs = []
            for k in range(KW):
                idxs = jnp.minimum(k * L + iota, S - 1) * ROWS_P + r
                wvecs.append(plsc.load_gather(w_v, [idxs]))
            masks = [(k * L + iota) < S for k in range(KW)]
            mvec = jnp.where(masks[0], wvecs[0], -1e30)
            for k in range(1, KW):
                mvec = jnp.maximum(mvec, jnp.where(masks[k], wvecs[k], -1e30))
            mx = jnp.max(mvec)
            svec = jnp.zeros((L,), jnp.float32)
            evecs = []
            for k in range(KW):
                e_k = jnp.where(masks[k], jnp.exp(wvecs[k] - mx), 0.0)
                evecs.append(e_k)
                svec = svec + e_k
            inv = jnp.ones((L,), jnp.float32) / lax.broadcast(jnp.sum(svec), (L,))
            wbase = r * (KW * L)
            for k in range(KW):
                wexp_v[pl.ds(wbase + k * L, L)] = evecs[k] * inv
            acc0 = jnp.zeros((L,), jnp.float32)
            acc1 = jnp.zeros((L,), jnp.float32)
            for j in range(S):
                wb = plsc.load_gather(wexp_v, [lax.broadcast(wbase + j, (L,))])
                acc0 = acc0 + wb * emb_v[j * ROWS_P + r, pl.ds(0, L)]
                acc1 = acc1 + wb * emb_v[j * ROWS_P + r, pl.ds(L, L)]
            out_v[r, pl.ds(0, L)] = acc0
            out_v[r, pl.ds(L, L)] = acc1
            return _

        lax.fori_loop(0, ROWS_P, row_body, 0)

        pltpu.sync_copy(out_v, out_hbm.at[pl.ds(col0 + p * ROWS_P, ROWS_P), :])


@jax.jit
def _run(desc_t, word_rm, weight_flat):
    mesh = plsc.VectorSubcoreMesh(core_axis_name="c", subcore_axis_name="s")
    return pl.kernel(
        _body,
        out_type=jax.ShapeDtypeStruct((B, D), jnp.float32),
        mesh=mesh,
        scratch_types=[
            pltpu.VMEM((S, ROWS_W), jnp.int32),
            pltpu.VMEM((TOK_P, D), jnp.float32),
            pltpu.VMEM((TOK_P,), jnp.float32),
            pltpu.VMEM((ROWS_P * KW * L,), jnp.float32),
            pltpu.VMEM((ROWS_P, D), jnp.float32),
            pltpu.SemaphoreType.DMA,
            pltpu.SemaphoreType.DMA,
        ],
        compiler_params=pltpu.CompilerParams(
            needs_layout_passes=False, use_tc_tiling_on_sc=False),
    )(desc_t, word_rm, weight_flat)


def kernel(desc, word_table, weight_table):
    tail_flat = word_table[TAIL_V0:, :].reshape(TAIL_N * D)
    wt_flat = _transpose_run(word_table.T, tail_flat)
    wt_rm = wt_flat.reshape(VOCAB, D)
    return _run(desc.T, wt_rm, weight_table.reshape(VOCAB))


# speedup vs baseline: 1.1966x; 1.0019x over previous
import functools

import jax
import jax.numpy as jnp
from jax import lax
from jax.experimental import pallas as pl
from jax.experimental.pallas import tpu as pltpu
from jax.experimental.pallas import tpu_sc as plsc

VOCAB = 1000000
D = 32
B = 4096
S = 50
L = 16
NC, NS = 2, 16
NW = NC * NS
ROWS_W = B // NW
ROWS_P = 64
NPASS = ROWS_W // ROWS_P
TOK_P = ROWS_P * S
KW = (S + L - 1) // L

TILE_V = 128
NVT = 999936 // TILE_V
G = 4
NB_ALL = NVT // G
NSLOT = 61
TAIL_V0 = NVT * TILE_V
TAIL_N = VOCAB - TAIL_V0
BATCH_W = G * TILE_V * D


def _tbody(word_t, tail_flat, out_flat, tin_a, tin_b, tout_a, tout_b,
           isem, osem):
    wid = lax.axis_index("s") * NC + lax.axis_index("c")
    iota = lax.iota(jnp.int32, L)

    @pl.when(wid == 0)
    def _():
        pltpu.sync_copy(tail_flat, tout_a.at[pl.ds(0, TAIL_N * D)])
        pltpu.sync_copy(tout_a.at[pl.ds(0, TAIL_N * D)],
                        out_flat.at[pl.ds(TAIL_V0 * D, TAIL_N * D)])

    def fire_in(slot, tin):
        v0 = (wid + NW * slot) * (G * TILE_V)
        pltpu.async_copy(
            word_t.at[:, pl.ds(v0, G * TILE_V)], tin, isem)

    def wait_in(tin):
        pltpu.make_async_copy(
            word_t.at[:, pl.ds(0, G * TILE_V)], tin, isem).wait()

    cvec0 = iota * (G * TILE_V)
    cvec1 = (iota + L) * (G * TILE_V)
    zerov = jnp.zeros((L,), jnp.int32)

    def transpose(tin, tout):
        @plsc.parallel_loop(0, G * 32, unroll=4)
        def _(tq):
            base = tq * TILE_V
            colb = lax.shift_left(lax.shift_right_logical(tq, 5), 7) \
                + lax.shift_left(jnp.bitwise_and(tq, 31), 2)
            for j in range(4):
                cols = lax.broadcast(colb + j, (L,))
                for c0 in (0, 16):
                    g = plsc.load_gather(tin, [c0 + iota, cols])
                    tout[pl.ds(base + j * D + c0, L)] = g

    def fire_out(slot, tout):
        bi = wid + NW * slot
        pltpu.async_copy(tout, out_flat.at[pl.ds(bi * BATCH_W, BATCH_W)], osem)

    def wait_out(tout):
        pltpu.make_async_copy(
            out_flat.at[pl.ds(0, BATCH_W)], tout, osem).wait()

    fire_in(0, tin_a)

    def step(k, carry):
        slot_a = 2 * k
        slot_b = 2 * k + 1
        fire_in(slot_b, tin_b)
        wait_in(tin_a)

        @pl.when(k > 0)
        def _():
            wait_out(tout_a)

        transpose(tin_a, tout_a)
        fire_out(slot_a, tout_a)
        fire_in(slot_a + 2, tin_a)
        wait_in(tin_b)

        @pl.when(k > 0)
        def _():
            wait_out(tout_b)

        transpose(tin_b, tout_b)
        fire_out(slot_b, tout_b)
        return carry

    lax.fori_loop(0, (NSLOT - 1) // 2, step, 0)

    wait_in(tin_a)
    wait_out(tout_a)
    transpose(tin_a, tout_a)
    fire_out(NSLOT - 1, tout_a)

    @pl.when(wid == NW - 1)
    def _():
        wait_out(tout_b)
        v0 = (NB_ALL - 1) * (G * TILE_V)
        pltpu.async_copy(
            word_t.at[:, pl.ds(v0, G * TILE_V)], tin_b, isem)
        wait_in(tin_b)
        transpose(tin_b, tout_b)
        pltpu.sync_copy(tout_b,
                        out_flat.at[pl.ds((NB_ALL - 1) * BATCH_W, BATCH_W)])

    wait_out(tout_a)
    @pl.when(wid != NW - 1)
    def _():
        wait_out(tout_b)


@jax.jit
def _transpose_run(word_t, tail_flat):
    mesh = plsc.VectorSubcoreMesh(core_axis_name="c", subcore_axis_name="s")
    return pl.kernel(
        _tbody,
        out_type=jax.ShapeDtypeStruct((VOCAB * D,), jnp.float32),
        mesh=mesh,
        scratch_types=[
            pltpu.VMEM((32, G * TILE_V), jnp.float32),
            pltpu.VMEM((32, G * TILE_V), jnp.float32),
            pltpu.VMEM((BATCH_W,), jnp.float32),
            pltpu.VMEM((BATCH_W,), jnp.float32),
            pltpu.SemaphoreType.DMA,
            pltpu.SemaphoreType.DMA,
        ],
        compiler_params=pltpu.CompilerParams(
            needs_layout_passes=False, use_tc_tiling_on_sc=True),
    )(word_t, tail_flat)


def _body(desc_t, word_hbm, weight_hbm, out_hbm,
          idx_v, emb_v, w_v, wexp_v, out_v, gsem, wsem):
    wid = lax.axis_index("s") * NC + lax.axis_index("c")
    iota = lax.iota(jnp.int32, L)
    col0 = wid * ROWS_W

    pltpu.sync_copy(desc_t.at[:, pl.ds(col0, ROWS_W)], idx_v)

    for p in range(NPASS):
        copies = []
        for j in range(S):
            ids = idx_v.at[j, pl.ds(p * ROWS_P, ROWS_P)]
            copies.append(pltpu.async_copy(
                word_hbm.at[ids], emb_v.at[pl.ds(j * ROWS_P, ROWS_P), :], gsem))
            copies.append(pltpu.async_copy(
                weight_hbm.at[ids], w_v.at[pl.ds(j * ROWS_P, ROWS_P)], wsem))
        for c in copies:
            c.wait()

        def row_body(r, _):
            wvecs = []
            for k in range(KW):
                idxs = jnp.minimum(k * L + iota, S - 1) * ROWS_P + r
                wvecs.append(plsc.load_gather(w_v, [idxs]))
            masks = [(k * L + iota) < S for k in range(KW)]
            mvec = jnp.where(masks[0], wvecs[0], -1e30)
            for k in range(1, KW):
                mvec = jnp.maximum(mvec, jnp.where(masks[k], wvecs[k], -1e30))
            mx = jnp.max(mvec)
            svec = jnp.zeros((L,), jnp.float32)
            evecs = []
            for k in range(KW):
                e_k = jnp.where(masks[k], jnp.exp(wvecs[k] - mx), 0.0)
                evecs.append(e_k)
                svec = svec + e_k
            inv = jnp.ones((L,), jnp.float32) / lax.broadcast(jnp.sum(svec), (L,))
            wbase = r * (KW * L)
            for k in range(KW):
                wexp_v[pl.ds(wbase + k * L, L)] = evecs[k] * inv
            acc0 = jnp.zeros((L,), jnp.float32)
            acc1 = jnp.zeros((L,), jnp.float32)
            for j in range(S):
                wb = plsc.load_gather(wexp_v, [lax.broadcast(wbase + j, (L,))])
                acc0 = acc0 + wb * emb_v[j * ROWS_P + r, pl.ds(0, L)]
                acc1 = acc1 + wb * emb_v[j * ROWS_P + r, pl.ds(L, L)]
            out_v[r, pl.ds(0, L)] = acc0
            out_v[r, pl.ds(L, L)] = acc1
            return _

        lax.fori_loop(0, ROWS_P, row_body, 0)

        pltpu.sync_copy(out_v, out_hbm.at[pl.ds(col0 + p * ROWS_P, ROWS_P), :])


@jax.jit
def _run(desc_t, word_rm, weight_flat):
    mesh = plsc.VectorSubcoreMesh(core_axis_name="c", subcore_axis_name="s")
    return pl.kernel(
        _body,
        out_type=jax.ShapeDtypeStruct((B, D), jnp.float32),
        mesh=mesh,
        scratch_types=[
            pltpu.VMEM((S, ROWS_W), jnp.int32),
            pltpu.VMEM((TOK_P, D), jnp.float32),
            pltpu.VMEM((TOK_P,), jnp.float32),
            pltpu.VMEM((ROWS_P * KW * L,), jnp.float32),
            pltpu.VMEM((ROWS_P, D), jnp.float32),
            pltpu.SemaphoreType.DMA,
            pltpu.SemaphoreType.DMA,
        ],
        compiler_params=pltpu.CompilerParams(
            needs_layout_passes=False, use_tc_tiling_on_sc=False),
    )(desc_t, word_rm, weight_flat)


def kernel(desc, word_table, weight_table):
    tail_flat = word_table[TAIL_V0:, :].reshape(TAIL_N * D)
    wt_flat = _transpose_run(word_table.T, tail_flat)
    wt_rm = wt_flat.reshape(VOCAB, D)
    return _run(desc.T, wt_rm, weight_table.reshape(VOCAB))
